# baseline (speedup 1.0000x reference)
import jax
import jax.numpy as jnp
from jax import lax
from jax.experimental import pallas as pl
from jax.experimental.pallas import tpu as pltpu
from jax.experimental.pallas import tpu_sc as plsc

B = 16384
D = 64
H = 8
L = 16

NC = 2
NS = 16
NW = NC * NS
BPW = B // NW
IDX_CHUNK = 128
NCHUNK = BPW // IDX_CHUNK

ROWCHUNK = 64
NGROUP = ROWCHUNK // L
NROWCHUNK = BPW // ROWCHUNK

P_B1 = D * 3 * H * L
P_W2 = P_B1 + H * L
P_B2 = P_W2 + H * L
P_TOT = P_B2 + L


def _fused_body(uidx_h, iidx_h, utab_h, itab_h, pars_h, y_h,
                uidx_v, iidx_v, urows, irows, pars_v, yv, sem):
  wid = lax.axis_index("s") * NC + lax.axis_index("c")
  base = wid * BPW
  pltpu.sync_copy(pars_h, pars_v)
  pltpu.sync_copy(uidx_h.at[pl.ds(base, BPW)], uidx_v)
  pltpu.sync_copy(iidx_h.at[pl.ds(base, BPW)], iidx_v)
  copies = []
  for j in range(NCHUNK):
    sl = pl.ds(j * IDX_CHUNK, IDX_CHUNK)
    copies.append(pltpu.async_copy(utab_h.at[uidx_v.at[sl]], urows.at[sl], sem))
    copies.append(pltpu.async_copy(itab_h.at[iidx_v.at[sl]], irows.at[sl], sem))
  for c in copies:
    c.wait()

  lane = lax.iota(jnp.int32, L)
  for rc in range(NROWCHUNK):
    rows0 = rc * ROWCHUNK
    rowvecs = [rows0 + g * L + lane for g in range(NGROUP)]

    def dbody(d, accs):
      accs = list(accs)
      wbase = d * (3 * H * L)
      wa = [pars_v[pl.ds(wbase + k * L, L)] for k in range(H)]
      wb = [pars_v[pl.ds(wbase + (H + k) * L, L)] for k in range(H)]
      wc = [pars_v[pl.ds(wbase + (2 * H + k) * L, L)] for k in range(H)]
      dcol = jnp.full((L,), d, jnp.int32)
      for g in range(NGROUP):
        u = plsc.load_gather(urows, [rowvecs[g], dcol])
        v = plsc.load_gather(irows, [rowvecs[g], dcol])
        e = u * v
        for k in range(H):
          accs[g * H + k] = accs[g * H + k] + e * wa[k] + u * wb[k] + v * wc[k]
      return tuple(accs)

    zero = jnp.zeros((L,), jnp.float32)
    accs = lax.fori_loop(0, D, dbody, tuple([zero] * (NGROUP * H)))

    for g in range(NGROUP):
      z = pars_v[pl.ds(P_B2, L)]
      for k in range(H):
        h = jnp.maximum(accs[g * H + k] + pars_v[pl.ds(P_B1 + k * L, L)], 0.0)
        z = z + h * pars_v[pl.ds(P_W2 + k * L, L)]
      y = 1.0 / (1.0 + jnp.exp(-z))
      yv[pl.ds(rows0 + g * L, L)] = y

  pltpu.sync_copy(yv, y_h.at[pl.ds(base, BPW)])


def _pack_params(W1, b1, W2, b2):
  wmix = jnp.stack([W1[0:D], W1[D:2 * D], W1[2 * D:3 * D]], axis=1)
  wmix = wmix.reshape(D * 3 * H, 1)
  parts = [
      jnp.broadcast_to(wmix, (D * 3 * H, L)).reshape(-1),
      jnp.broadcast_to(b1.reshape(H, 1), (H, L)).reshape(-1),
      jnp.broadcast_to(W2.reshape(H, 1), (H, L)).reshape(-1),
      jnp.broadcast_to(b2.reshape(1, 1), (1, L)).reshape(-1),
  ]
  return jnp.concatenate(parts)


@jax.jit
def kernel(group_inputs, user_inputs, item_inputs, user_table, item_table, W1, b1, W2, b2):
  del group_inputs
  pars = _pack_params(W1, b1, W2, b2)
  mesh = plsc.VectorSubcoreMesh(core_axis_name="c", subcore_axis_name="s")
  f = pl.kernel(
      _fused_body,
      out_type=jax.ShapeDtypeStruct((B,), jnp.float32),
      mesh=mesh,
      scratch_types=[
          pltpu.VMEM((BPW,), jnp.int32),
          pltpu.VMEM((BPW,), jnp.int32),
          pltpu.VMEM((BPW, D), jnp.float32),
          pltpu.VMEM((BPW, D), jnp.float32),
          pltpu.VMEM((P_TOT,), jnp.float32),
          pltpu.VMEM((BPW,), jnp.float32),
          pltpu.SemaphoreType.DMA,
      ],
      compiler_params=pltpu.CompilerParams(use_tc_tiling_on_sc=False,
                                           needs_layout_passes=False),
  )
  y = f(user_inputs.astype(jnp.int32), item_inputs.astype(jnp.int32),
        user_table, item_table, pars)
  return y.reshape(B, 1)


# speedup vs baseline: 1.3397x; 1.3397x over previous
import jax
import jax.numpy as jnp
from jax import lax
from jax.experimental import pallas as pl
from jax.experimental.pallas import tpu as pltpu
from jax.experimental.pallas import tpu_sc as plsc

B = 16384
D = 64

NC = 2
NS = 16
NW = NC * NS
BPW = B // NW
IDX_CHUNK = 128
HALF = 256
NHALF = BPW // HALF

BLK = 2048


def _gather_body(uidx_h, iidx_h, utab_h, itab_h, u_out, i_out,
                 uidx_v, iidx_v, ubuf, ibuf, sem):
  wid = lax.axis_index("s") * NC + lax.axis_index("c")
  base = wid * BPW
  pltpu.sync_copy(uidx_h.at[pl.ds(base, BPW)], uidx_v)
  pltpu.sync_copy(iidx_h.at[pl.ds(base, BPW)], iidx_v)
  for half in range(NHALF):
    copies = []
    for j in range(HALF // IDX_CHUNK):
      isl = pl.ds(half * HALF + j * IDX_CHUNK, IDX_CHUNK)
      dsl = pl.ds(j * IDX_CHUNK, IDX_CHUNK)
      copies.append(pltpu.async_copy(utab_h.at[uidx_v.at[isl]], ubuf.at[dsl], sem))
      copies.append(pltpu.async_copy(itab_h.at[iidx_v.at[isl]], ibuf.at[dsl], sem))
    for c in copies:
      c.wait()
    osl = pl.ds(base + half * HALF, HALF)
    pltpu.sync_copy(ubuf, u_out.at[osl])
    pltpu.sync_copy(ibuf, i_out.at[osl])


def _sc_gather(uidx_half, iidx_half, utab2, itab2):
  mesh = plsc.VectorSubcoreMesh(core_axis_name="c", subcore_axis_name="s")
  f = pl.kernel(
      _gather_body,
      out_type=(
          jax.ShapeDtypeStruct((B, 128), jnp.float32),
          jax.ShapeDtypeStruct((B, 128), jnp.float32),
      ),
      mesh=mesh,
      scratch_types=[
          pltpu.VMEM((BPW,), jnp.int32),
          pltpu.VMEM((BPW,), jnp.int32),
          pltpu.VMEM((HALF, 128), jnp.float32),
          pltpu.VMEM((HALF, 128), jnp.float32),
          pltpu.SemaphoreType.DMA,
      ],
  )
  return f(uidx_half, iidx_half, utab2, itab2)


def _mlp_body(u2_ref, i2_ref, pu_ref, pi_ref, wa_ref, wb_ref, wc_ref,
              b1_ref, w2t_ref, b2_ref, o_ref):
  u2 = u2_ref[...]
  i2 = i2_ref[...]
  u = jnp.where(pu_ref[...] > 0.5, u2[:, D:2 * D], u2[:, 0:D])
  v = jnp.where(pi_ref[...] > 0.5, i2[:, D:2 * D], i2[:, 0:D])
  e = u * v
  h = (jnp.dot(e, wa_ref[...], preferred_element_type=jnp.float32)
       + jnp.dot(u, wb_ref[...], preferred_element_type=jnp.float32)
       + jnp.dot(v, wc_ref[...], preferred_element_type=jnp.float32)
       + b1_ref[...])
  h = jnp.maximum(h, 0.0)
  z = jnp.sum(h * w2t_ref[...], axis=1, keepdims=True) + b2_ref[...]
  o_ref[...] = jax.nn.sigmoid(z)


def _tc_mlp(u2, i2, pu, pi, W1, b1, W2, b2):
  wa = W1[0:D]
  wb = W1[D:2 * D]
  wc = W1[2 * D:3 * D]
  b1r = b1.reshape(1, 8)
  w2t = W2.reshape(1, 8)
  b2r = b2.reshape(1, 1)
  grid = (B // BLK,)
  return pl.pallas_call(
      _mlp_body,
      grid=grid,
      in_specs=[
          pl.BlockSpec((BLK, 128), lambda n: (n, 0)),
          pl.BlockSpec((BLK, 128), lambda n: (n, 0)),
          pl.BlockSpec((BLK, 1), lambda n: (n, 0)),
          pl.BlockSpec((BLK, 1), lambda n: (n, 0)),
          pl.BlockSpec((D, 8), lambda n: (0, 0)),
          pl.BlockSpec((D, 8), lambda n: (0, 0)),
          pl.BlockSpec((D, 8), lambda n: (0, 0)),
          pl.BlockSpec((1, 8), lambda n: (0, 0)),
          pl.BlockSpec((1, 8), lambda n: (0, 0)),
          pl.BlockSpec((1, 1), lambda n: (0, 0)),
      ],
      out_specs=pl.BlockSpec((BLK, 1), lambda n: (n, 0)),
      out_shape=jax.ShapeDtypeStruct((B, 1), jnp.float32),
      compiler_params=pltpu.CompilerParams(
          dimension_semantics=("arbitrary",),
      ),
  )(u2, i2, pu, pi, wa, wb, wc, b1r, w2t, b2r)


@jax.jit
def kernel(group_inputs, user_inputs, item_inputs, user_table, item_table, W1, b1, W2, b2):
  del group_inputs
  ui = user_inputs.astype(jnp.int32)
  ii = item_inputs.astype(jnp.int32)
  utab2 = user_table.reshape(50000, 128)
  itab2 = item_table.reshape(50000, 128)
  u2, i2 = _sc_gather(ui // 2, ii // 2, utab2, itab2)
  pu = (ui % 2).astype(jnp.float32).reshape(B, 1)
  pi = (ii % 2).astype(jnp.float32).reshape(B, 1)
  return _tc_mlp(u2, i2, pu, pi, W1, b1, W2, b2)
